# baseline (speedup 1.0000x reference)
import functools

import jax
import jax.numpy as jnp
from jax import lax
from jax.experimental import pallas as pl
from jax.experimental.pallas import tpu as pltpu
from jax.experimental.pallas import tpu_sc as plsc

_B = 64
_V = 1000000

_CH = 2688
_NVEC = _CH // 16
_KPT = 2
_SC_SPAN = 32 * _KPT * _CH
_NPAIR = 8 * _KPT // 2

_TC_VC = 14336
_TC_OFF = _SC_SPAN // _TC_VC
_TC_NCHUNK = (_V - _SC_SPAN + _TC_VC - 1) // _TC_VC

_LN2 = 0.6931471805599453
_PF = [
    -6.07475245e-03, 3.44179115e-02, -9.23123095e-02, 1.64781887e-01,
    -2.39189722e-01, 3.31333659e-01, -4.99801099e-01, 9.99991449e-01,
    9.09903356e-08,
]

_NEG_INF = float("-inf")

_DNUMS = lax.GatherDimensionNumbers(
    offset_dims=(), collapsed_slice_dims=(0,), start_index_map=(0,))


def _lane_perm(x, perm):
    return lax.gather(x, perm, _DNUMS, slice_sizes=(1,),
                      mode=lax.GatherScatterMode.PROMISE_IN_BOUNDS)


def _lane_allreduce(val, idx, perms):
    for perm in perms:
        ov = _lane_perm(val, perm)
        oi = _lane_perm(idx, perm)
        better = (ov > val) | ((ov == val) & (oi < idx))
        val = jnp.where(better, ov, val)
        idx = jnp.where(better, oi, idx)
    return val, idx


def _key16(l, n, t):
    x = n + 1e-10
    bits = lax.bitcast_convert_type(x, jnp.int32)
    ef = lax.shift_right_logical(bits, 23).astype(jnp.float32) - 127.0
    m = lax.bitcast_convert_type(
        (bits & 0x007FFFFF) | 0x3F800000, jnp.float32)
    d = m - 1.0
    acc = jnp.full((16,), _PF[0], jnp.float32)
    for cc in _PF[1:]:
        acc = acc * d + cc
    lnx = ef * _LN2 + acc
    return l - t * lnx


def _sc_body(logits, noise, temps16, outv, outi,
             lbufA, nbufA, lbufB, nbufB, tvm, resv, resi, semA, semB):
    c = lax.axis_index("c")
    s = lax.axis_index("s")
    wid = s * 2 + c
    iota = lax.iota(jnp.int32, 16)
    perms = [jnp.reshape((iota + sh) & 15, (16, 1)) for sh in (8, 4, 2, 1)]

    pltpu.sync_copy(temps16, tvm)

    def _decode(i):
        rb = lax.div(i, _KPT)
        k = lax.rem(i, _KPT)
        chunk = wid + 32 * k
        start = pl.multiple_of(chunk * _CH, 128)
        row0 = pl.multiple_of(rb * 8, 8)
        return rb, row0, start

    def _issue(row0, start, lb, nb, sem):
        pltpu.make_async_copy(
            logits.at[pl.ds(row0, 8), pl.ds(start, _CH)], lb, sem).start()
        pltpu.make_async_copy(
            noise.at[pl.ds(row0, 8), pl.ds(start, _CH)], nb, sem).start()

    def _wait(row0, start, lb, nb, sem):
        pltpu.make_async_copy(
            logits.at[pl.ds(row0, 8), pl.ds(start, _CH)], lb, sem).wait()
        pltpu.make_async_copy(
            noise.at[pl.ds(row0, 8), pl.ds(start, _CH)], nb, sem).wait()

    def _proc_item(lb, nb, start, rb, row0, gval, gidx):
        half = (rb & 1) * 8
        for rr in range(8):
            tvec = tvm[pl.ds((row0 + rr) * 16, 16)]

            def vbody(j, carry, _rr=rr, _t=tvec):
                vmax, vidx = carry
                key = _key16(lb[_rr, pl.ds(j * 16, 16)],
                             nb[_rr, pl.ds(j * 16, 16)], _t)
                msk = key > vmax
                vmax = jnp.where(msk, key, vmax)
                vidx = jnp.where(msk, jnp.broadcast_to(j, (16,)), vidx)
                return vmax, vidx

            vmax, vidx = lax.fori_loop(
                0, _NVEC, vbody,
                (jnp.full((16,), _NEG_INF, jnp.float32),
                 jnp.zeros((16,), jnp.int32)),
                unroll=4)
            col = start + vidx * 16 + iota
            nv, ni = _lane_allreduce(vmax, col, perms)
            lmask = iota == (half + rr)
            better = lmask & ((nv > gval) | ((nv == gval) & (ni < gidx)))
            gval = jnp.where(better, nv, gval)
            gidx = jnp.where(better, ni, gidx)
        return gval, gidx

    rb0, row00, st00 = _decode(0)
    _issue(row00, st00, lbufA, nbufA, semA)

    def pair_body(p, carry):
        gval, gidx = carry
        rbA, rowA, stA = _decode(2 * p)
        rbB, rowB, stB = _decode(2 * p + 1)

        _issue(rowB, stB, lbufB, nbufB, semB)
        _wait(rowA, stA, lbufA, nbufA, semA)
        gval, gidx = _proc_item(lbufA, nbufA, stA, rbA, rowA, gval, gidx)

        @pl.when(p + 1 < _NPAIR)
        def _():
            _, rowN, stN = _decode(2 * p + 2)
            _issue(rowN, stN, lbufA, nbufA, semA)

        _wait(rowB, stB, lbufB, nbufB, semB)
        gval, gidx = _proc_item(lbufB, nbufB, stB, rbB, rowB, gval, gidx)

        flush = lax.rem(p, _KPT) == _KPT - 1
        gbase = lax.div(p, _KPT) * 16

        @pl.when(flush)
        def _():
            resv[pl.ds(gbase, 16)] = gval
            resi[pl.ds(gbase, 16)] = gidx

        gval = jnp.where(flush, jnp.full((16,), _NEG_INF, jnp.float32), gval)
        gidx = jnp.where(flush, jnp.zeros((16,), jnp.int32), gidx)
        return gval, gidx

    lax.fori_loop(
        0, _NPAIR, pair_body,
        (jnp.full((16,), _NEG_INF, jnp.float32), jnp.zeros((16,), jnp.int32)),
    )

    obase = pl.multiple_of(wid * _B, 8)
    pltpu.sync_copy(resv, outv.at[pl.ds(obase, _B)])
    pltpu.sync_copy(resi, outi.at[pl.ds(obase, _B)])


_sc_scan = functools.partial(
    pl.kernel,
    mesh=plsc.VectorSubcoreMesh(core_axis_name="c", subcore_axis_name="s"),
    out_type=[
        jax.ShapeDtypeStruct((32 * _B,), jnp.float32),
        jax.ShapeDtypeStruct((32 * _B,), jnp.int32),
    ],
    scratch_types=[
        pltpu.VMEM((8, _CH), jnp.float32),
        pltpu.VMEM((8, _CH), jnp.float32),
        pltpu.VMEM((8, _CH), jnp.float32),
        pltpu.VMEM((8, _CH), jnp.float32),
        pltpu.VMEM((_B * 16,), jnp.float32),
        pltpu.VMEM((_B,), jnp.float32),
        pltpu.VMEM((_B,), jnp.int32),
        pltpu.SemaphoreType.DMA,
        pltpu.SemaphoreType.DMA,
    ],
)(_sc_body)


def _tc_body(temp_ref, logits_ref, noise_ref, ov_ref, oi_ref,
             best_val, best_idx):
    i = pl.program_id(0)
    temp = temp_ref[...]
    logits = logits_ref[...]
    noise = noise_ref[...]
    col = (jax.lax.broadcasted_iota(jnp.int32, (_B, _TC_VC), 1)
           + (i + _TC_OFF) * _TC_VC)
    valid = col < _V
    key = logits - temp * jnp.log(noise + 1e-10)
    key = jnp.where(valid, key, -jnp.inf)
    local_max = jnp.max(key, axis=1, keepdims=True)
    at_max = (key == local_max) & valid
    local_idx = jnp.min(jnp.where(at_max, col, _V), axis=1, keepdims=True)

    @pl.when(i == 0)
    def _():
        best_val[...] = local_max
        best_idx[...] = local_idx

    @pl.when(i > 0)
    def _():
        bv = best_val[...]
        take = local_max > bv
        best_val[...] = jnp.where(take, local_max, bv)
        best_idx[...] = jnp.where(take, local_idx, best_idx[...])

    @pl.when(i == _TC_NCHUNK - 1)
    def _():
        ov_ref[...] = best_val[...]
        oi_ref[...] = best_idx[...]


def _merge_body(v_ref, i_ref, tv_ref, ti_ref, o_ref):
    v = v_ref[...]
    i = i_ref[...]
    tv = tv_ref[...]
    ti = ti_ref[...]
    sv = jnp.max(v, axis=1, keepdims=True)
    si = jnp.min(jnp.where(v == sv, i, _V), axis=1, keepdims=True)
    take = (tv > sv) | ((tv == sv) & (ti < si))
    o_ref[...] = jnp.where(take, ti, si)


@jax.jit
def kernel(logits, temperatures, exp_noise):
    l = logits.astype(jnp.float32)
    t = temperatures.astype(jnp.float32)
    t16 = jnp.broadcast_to(t[:, None], (_B, 16)).reshape(_B * 16)
    pv, pi = _sc_scan(l, exp_noise, t16)
    tv, ti = pl.pallas_call(
        _tc_body,
        grid=(_TC_NCHUNK,),
        in_specs=[
            pl.BlockSpec((_B, 1), lambda i: (0, 0)),
            pl.BlockSpec((_B, _TC_VC), lambda i: (0, i)),
            pl.BlockSpec((_B, _TC_VC), lambda i: (0, i)),
        ],
        out_specs=[
            pl.BlockSpec((_B, 1), lambda i: (0, 0)),
            pl.BlockSpec((_B, 1), lambda i: (0, 0)),
        ],
        out_shape=[
            jax.ShapeDtypeStruct((_B, 1), jnp.float32),
            jax.ShapeDtypeStruct((_B, 1), jnp.int32),
        ],
        scratch_shapes=[
            pltpu.VMEM((_B, 1), jnp.float32),
            pltpu.VMEM((_B, 1), jnp.int32),
        ],
    )(t.reshape(_B, 1), l, exp_noise)
    out = pl.pallas_call(
        _merge_body,
        out_shape=jax.ShapeDtypeStruct((_B, 1), jnp.int32),
    )(pv.reshape(32, _B).T, pi.reshape(32, _B).T, tv, ti)
    return out.reshape(_B)


# speedup vs baseline: 1.0013x; 1.0013x over previous
import functools

import jax
import jax.numpy as jnp
from jax import lax
from jax.experimental import pallas as pl
from jax.experimental.pallas import tpu as pltpu
from jax.experimental.pallas import tpu_sc as plsc

_B = 64
_V = 1000000

_CH = 2688
_NVEC = _CH // 16
_KPT = 2
_SC_SPAN = 32 * _KPT * _CH
_NPAIR = 8 * _KPT // 2

_TC_VC = 14336
_TC_OFF = _SC_SPAN // _TC_VC
_TC_NCHUNK = (_V - _SC_SPAN + _TC_VC - 1) // _TC_VC

_LN2 = 0.6931471805599453
_PF = [
    -6.07475245e-03, 3.44179115e-02, -9.23123095e-02, 1.64781887e-01,
    -2.39189722e-01, 3.31333659e-01, -4.99801099e-01, 9.99991449e-01,
    9.09903356e-08,
]

_NEG_INF = float("-inf")

_DNUMS = lax.GatherDimensionNumbers(
    offset_dims=(), collapsed_slice_dims=(0,), start_index_map=(0,))


def _lane_perm(x, perm):
    return lax.gather(x, perm, _DNUMS, slice_sizes=(1,),
                      mode=lax.GatherScatterMode.PROMISE_IN_BOUNDS)


def _lane_allreduce(val, idx, perms):
    for perm in perms:
        ov = _lane_perm(val, perm)
        oi = _lane_perm(idx, perm)
        better = (ov > val) | ((ov == val) & (oi < idx))
        val = jnp.where(better, ov, val)
        idx = jnp.where(better, oi, idx)
    return val, idx


def _key16(l, n, t):
    x = n + 1e-10
    bits = lax.bitcast_convert_type(x, jnp.int32)
    ef = lax.shift_right_logical(bits, 23).astype(jnp.float32) - 127.0
    m = lax.bitcast_convert_type(
        (bits & 0x007FFFFF) | 0x3F800000, jnp.float32)
    d = m - 1.0
    acc = jnp.full((16,), _PF[0], jnp.float32)
    for cc in _PF[1:]:
        acc = acc * d + cc
    lnx = ef * _LN2 + acc
    return l - t * lnx


def _sc_body(logits, noise, temps16, outv, outi,
             lbufA, nbufA, lbufB, nbufB, tvm, resv, resi, semA, semB):
    c = lax.axis_index("c")
    s = lax.axis_index("s")
    wid = s * 2 + c
    iota = lax.iota(jnp.int32, 16)
    perms = [jnp.reshape((iota + sh) & 15, (16, 1)) for sh in (8, 4, 2, 1)]

    pltpu.sync_copy(temps16, tvm)

    def _decode(i):
        rb = lax.div(i, _KPT)
        k = lax.rem(i, _KPT)
        chunk = wid + 32 * k
        start = pl.multiple_of(chunk * _CH, 128)
        row0 = pl.multiple_of(rb * 8, 8)
        return rb, row0, start

    def _issue(row0, start, lb, nb, sem):
        pltpu.make_async_copy(
            logits.at[pl.ds(row0, 8), pl.ds(start, _CH)], lb, sem).start()
        pltpu.make_async_copy(
            noise.at[pl.ds(row0, 8), pl.ds(start, _CH)], nb, sem).start()

    def _wait(row0, start, lb, nb, sem):
        pltpu.make_async_copy(
            logits.at[pl.ds(row0, 8), pl.ds(start, _CH)], lb, sem).wait()
        pltpu.make_async_copy(
            noise.at[pl.ds(row0, 8), pl.ds(start, _CH)], nb, sem).wait()

    def _proc_item(lb, nb, start, rb, row0, gval, gidx):
        half = (rb & 1) * 8
        for rr in range(8):
            tvec = tvm[pl.ds((row0 + rr) * 16, 16)]

            def vbody(j, carry, _rr=rr, _t=tvec):
                vmax, vidx = carry
                key = _key16(lb[_rr, pl.ds(j * 16, 16)],
                             nb[_rr, pl.ds(j * 16, 16)], _t)
                msk = key > vmax
                vmax = jnp.where(msk, key, vmax)
                vidx = jnp.where(msk, jnp.broadcast_to(j, (16,)), vidx)
                return vmax, vidx

            vmax, vidx = lax.fori_loop(
                0, _NVEC, vbody,
                (jnp.full((16,), _NEG_INF, jnp.float32),
                 jnp.zeros((16,), jnp.int32)),
                unroll=4)
            col = start + vidx * 16 + iota
            nv, ni = _lane_allreduce(vmax, col, perms)
            lmask = iota == (half + rr)
            better = lmask & ((nv > gval) | ((nv == gval) & (ni < gidx)))
            gval = jnp.where(better, nv, gval)
            gidx = jnp.where(better, ni, gidx)
        return gval, gidx

    rb0, row00, st00 = _decode(0)
    _issue(row00, st00, lbufA, nbufA, semA)

    def pair_body(p, carry):
        gval, gidx = carry
        rbA, rowA, stA = _decode(2 * p)
        rbB, rowB, stB = _decode(2 * p + 1)

        _issue(rowB, stB, lbufB, nbufB, semB)
        _wait(rowA, stA, lbufA, nbufA, semA)
        gval, gidx = _proc_item(lbufA, nbufA, stA, rbA, rowA, gval, gidx)

        @pl.when(p + 1 < _NPAIR)
        def _():
            _, rowN, stN = _decode(2 * p + 2)
            _issue(rowN, stN, lbufA, nbufA, semA)

        _wait(rowB, stB, lbufB, nbufB, semB)
        gval, gidx = _proc_item(lbufB, nbufB, stB, rbB, rowB, gval, gidx)

        flush = lax.rem(p, _KPT) == _KPT - 1
        gbase = lax.div(p, _KPT) * 16

        @pl.when(flush)
        def _():
            resv[pl.ds(gbase, 16)] = gval
            resi[pl.ds(gbase, 16)] = gidx

        gval = jnp.where(flush, jnp.full((16,), _NEG_INF, jnp.float32), gval)
        gidx = jnp.where(flush, jnp.zeros((16,), jnp.int32), gidx)
        return gval, gidx

    lax.fori_loop(
        0, _NPAIR, pair_body,
        (jnp.full((16,), _NEG_INF, jnp.float32), jnp.zeros((16,), jnp.int32)),
    )

    obase = pl.multiple_of(wid * _B, 8)
    pltpu.sync_copy(resv, outv.at[pl.ds(obase, _B)])
    pltpu.sync_copy(resi, outi.at[pl.ds(obase, _B)])


_sc_scan = functools.partial(
    pl.kernel,
    mesh=plsc.VectorSubcoreMesh(core_axis_name="c", subcore_axis_name="s"),
    out_type=[
        jax.ShapeDtypeStruct((32 * _B,), jnp.float32),
        jax.ShapeDtypeStruct((32 * _B,), jnp.int32),
    ],
    scratch_types=[
        pltpu.VMEM((8, _CH), jnp.float32),
        pltpu.VMEM((8, _CH), jnp.float32),
        pltpu.VMEM((8, _CH), jnp.float32),
        pltpu.VMEM((8, _CH), jnp.float32),
        pltpu.VMEM((_B * 16,), jnp.float32),
        pltpu.VMEM((_B,), jnp.float32),
        pltpu.VMEM((_B,), jnp.int32),
        pltpu.SemaphoreType.DMA,
        pltpu.SemaphoreType.DMA,
    ],
)(_sc_body)


def _tc_body(temp_ref, logits_ref, noise_ref, ov_ref, oi_ref,
             best_val, best_idx):
    i = pl.program_id(0)
    temp = temp_ref[...]
    logits = logits_ref[...]
    noise = noise_ref[...]
    col = (jax.lax.broadcasted_iota(jnp.int32, (_B, _TC_VC), 1)
           + (i + _TC_OFF) * _TC_VC)
    valid = col < _V
    key = logits - temp * jnp.log(noise + 1e-10)
    key = jnp.where(valid, key, -jnp.inf)
    local_max = jnp.max(key, axis=1, keepdims=True)
    at_max = (key == local_max) & valid
    local_idx = jnp.min(jnp.where(at_max, col, _V), axis=1, keepdims=True)

    @pl.when(i == 0)
    def _():
        best_val[...] = local_max
        best_idx[...] = local_idx

    @pl.when(i > 0)
    def _():
        bv = best_val[...]
        take = local_max > bv
        best_val[...] = jnp.where(take, local_max, bv)
        best_idx[...] = jnp.where(take, local_idx, best_idx[...])

    @pl.when(i == _TC_NCHUNK - 1)
    def _():
        ov_ref[...] = best_val[...]
        oi_ref[...] = best_idx[...]


def _merge_body(v_ref, i_ref, tv_ref, ti_ref, o_ref):
    v = v_ref[...]
    i = i_ref[...]
    tv = tv_ref[...]
    ti = ti_ref[...]
    sv = jnp.max(v, axis=1, keepdims=True)
    si = jnp.min(jnp.where(v == sv, i, _V), axis=1, keepdims=True)
    take = (tv > sv) | ((tv == sv) & (ti < si))
    o_ref[...] = jnp.where(take, ti, si)


@jax.jit
def kernel(logits, temperatures, exp_noise):
    l = logits.astype(jnp.float32)
    t = temperatures.astype(jnp.float32)
    t16 = jnp.broadcast_to(t[:, None], (_B, 16)).reshape(_B * 16)
    pv, pi = _sc_scan(l, exp_noise, t16)
    tv, ti = pl.pallas_call(
        _tc_body,
        grid=(_TC_NCHUNK,),
        in_specs=[
            pl.BlockSpec((_B, 1), lambda i: (0, 0)),
            pl.BlockSpec((_B, _TC_VC), lambda i: (0, i + _TC_OFF)),
            pl.BlockSpec((_B, _TC_VC), lambda i: (0, i + _TC_OFF)),
        ],
        out_specs=[
            pl.BlockSpec((_B, 1), lambda i: (0, 0)),
            pl.BlockSpec((_B, 1), lambda i: (0, 0)),
        ],
        out_shape=[
            jax.ShapeDtypeStruct((_B, 1), jnp.float32),
            jax.ShapeDtypeStruct((_B, 1), jnp.int32),
        ],
        scratch_shapes=[
            pltpu.VMEM((_B, 1), jnp.float32),
            pltpu.VMEM((_B, 1), jnp.int32),
        ],
    )(t.reshape(_B, 1), l, exp_noise)
    out = pl.pallas_call(
        _merge_body,
        out_shape=jax.ShapeDtypeStruct((_B, 1), jnp.int32),
    )(pv.reshape(32, _B).T, pi.reshape(32, _B).T, tv, ti)
    return out.reshape(_B)


# speedup vs baseline: 1.0793x; 1.0780x over previous
import functools

import jax
import jax.numpy as jnp
from jax import lax
from jax.experimental import pallas as pl
from jax.experimental.pallas import tpu as pltpu
from jax.experimental.pallas import tpu_sc as plsc

_B = 64
_V = 1000000

_CH = 2688
_NVEC = _CH // 16
_KPT = 2
_SC_SPAN = 32 * _KPT * _CH
_NPAIR = 8 * _KPT // 2

_TC_VC = 28672
_TC_OFF = _SC_SPAN // _TC_VC
_TC_NCHUNK = (_V - _SC_SPAN + _TC_VC - 1) // _TC_VC

_LN2 = 0.6931471805599453
_PF = [
    -6.07475245e-03, 3.44179115e-02, -9.23123095e-02, 1.64781887e-01,
    -2.39189722e-01, 3.31333659e-01, -4.99801099e-01, 9.99991449e-01,
    9.09903356e-08,
]

_NEG_INF = float("-inf")

_DNUMS = lax.GatherDimensionNumbers(
    offset_dims=(), collapsed_slice_dims=(0,), start_index_map=(0,))


def _lane_perm(x, perm):
    return lax.gather(x, perm, _DNUMS, slice_sizes=(1,),
                      mode=lax.GatherScatterMode.PROMISE_IN_BOUNDS)


def _lane_allreduce(val, idx, perms):
    for perm in perms:
        ov = _lane_perm(val, perm)
        oi = _lane_perm(idx, perm)
        better = (ov > val) | ((ov == val) & (oi < idx))
        val = jnp.where(better, ov, val)
        idx = jnp.where(better, oi, idx)
    return val, idx


def _key16(l, n, t):
    x = n + 1e-10
    bits = lax.bitcast_convert_type(x, jnp.int32)
    ef = lax.shift_right_logical(bits, 23).astype(jnp.float32) - 127.0
    m = lax.bitcast_convert_type(
        (bits & 0x007FFFFF) | 0x3F800000, jnp.float32)
    d = m - 1.0
    acc = jnp.full((16,), _PF[0], jnp.float32)
    for cc in _PF[1:]:
        acc = acc * d + cc
    lnx = ef * _LN2 + acc
    return l - t * lnx


def _sc_body(logits, noise, temps16, outv, outi,
             lbufA, nbufA, lbufB, nbufB, tvm, resv, resi, semA, semB):
    c = lax.axis_index("c")
    s = lax.axis_index("s")
    wid = s * 2 + c
    iota = lax.iota(jnp.int32, 16)
    perms = [jnp.reshape((iota + sh) & 15, (16, 1)) for sh in (8, 4, 2, 1)]

    pltpu.sync_copy(temps16, tvm)

    def _decode(i):
        rb = lax.div(i, _KPT)
        k = lax.rem(i, _KPT)
        chunk = wid + 32 * k
        start = pl.multiple_of(chunk * _CH, 128)
        row0 = pl.multiple_of(rb * 8, 8)
        return rb, row0, start

    def _issue(row0, start, lb, nb, sem):
        pltpu.make_async_copy(
            logits.at[pl.ds(row0, 8), pl.ds(start, _CH)], lb, sem).start()
        pltpu.make_async_copy(
            noise.at[pl.ds(row0, 8), pl.ds(start, _CH)], nb, sem).start()

    def _wait(row0, start, lb, nb, sem):
        pltpu.make_async_copy(
            logits.at[pl.ds(row0, 8), pl.ds(start, _CH)], lb, sem).wait()
        pltpu.make_async_copy(
            noise.at[pl.ds(row0, 8), pl.ds(start, _CH)], nb, sem).wait()

    def _proc_item(lb, nb, start, rb, row0, gval, gidx):
        half = (rb & 1) * 8
        for rr in range(8):
            tvec = tvm[pl.ds((row0 + rr) * 16, 16)]

            def vbody(j, carry, _rr=rr, _t=tvec):
                vmax, vidx = carry
                key = _key16(lb[_rr, pl.ds(j * 16, 16)],
                             nb[_rr, pl.ds(j * 16, 16)], _t)
                msk = key > vmax
                vmax = jnp.where(msk, key, vmax)
                vidx = jnp.where(msk, jnp.broadcast_to(j, (16,)), vidx)
                return vmax, vidx

            vmax, vidx = lax.fori_loop(
                0, _NVEC, vbody,
                (jnp.full((16,), _NEG_INF, jnp.float32),
                 jnp.zeros((16,), jnp.int32)),
                unroll=4)
            col = start + vidx * 16 + iota
            nv, ni = _lane_allreduce(vmax, col, perms)
            lmask = iota == (half + rr)
            better = lmask & ((nv > gval) | ((nv == gval) & (ni < gidx)))
            gval = jnp.where(better, nv, gval)
            gidx = jnp.where(better, ni, gidx)
        return gval, gidx

    rb0, row00, st00 = _decode(0)
    _issue(row00, st00, lbufA, nbufA, semA)

    def pair_body(p, carry):
        gval, gidx = carry
        rbA, rowA, stA = _decode(2 * p)
        rbB, rowB, stB = _decode(2 * p + 1)

        _issue(rowB, stB, lbufB, nbufB, semB)
        _wait(rowA, stA, lbufA, nbufA, semA)
        gval, gidx = _proc_item(lbufA, nbufA, stA, rbA, rowA, gval, gidx)

        @pl.when(p + 1 < _NPAIR)
        def _():
            _, rowN, stN = _decode(2 * p + 2)
            _issue(rowN, stN, lbufA, nbufA, semA)

        _wait(rowB, stB, lbufB, nbufB, semB)
        gval, gidx = _proc_item(lbufB, nbufB, stB, rbB, rowB, gval, gidx)

        flush = lax.rem(p, _KPT) == _KPT - 1
        gbase = lax.div(p, _KPT) * 16

        @pl.when(flush)
        def _():
            resv[pl.ds(gbase, 16)] = gval
            resi[pl.ds(gbase, 16)] = gidx

        gval = jnp.where(flush, jnp.full((16,), _NEG_INF, jnp.float32), gval)
        gidx = jnp.where(flush, jnp.zeros((16,), jnp.int32), gidx)
        return gval, gidx

    lax.fori_loop(
        0, _NPAIR, pair_body,
        (jnp.full((16,), _NEG_INF, jnp.float32), jnp.zeros((16,), jnp.int32)),
    )

    obase = pl.multiple_of(wid * _B, 8)
    pltpu.sync_copy(resv, outv.at[pl.ds(obase, _B)])
    pltpu.sync_copy(resi, outi.at[pl.ds(obase, _B)])


_sc_scan = functools.partial(
    pl.kernel,
    mesh=plsc.VectorSubcoreMesh(core_axis_name="c", subcore_axis_name="s"),
    out_type=[
        jax.ShapeDtypeStruct((32 * _B,), jnp.float32),
        jax.ShapeDtypeStruct((32 * _B,), jnp.int32),
    ],
    scratch_types=[
        pltpu.VMEM((8, _CH), jnp.float32),
        pltpu.VMEM((8, _CH), jnp.float32),
        pltpu.VMEM((8, _CH), jnp.float32),
        pltpu.VMEM((8, _CH), jnp.float32),
        pltpu.VMEM((_B * 16,), jnp.float32),
        pltpu.VMEM((_B,), jnp.float32),
        pltpu.VMEM((_B,), jnp.int32),
        pltpu.SemaphoreType.DMA,
        pltpu.SemaphoreType.DMA,
    ],
)(_sc_body)


def _tc_body(temp_ref, logits_ref, noise_ref, ov_ref, oi_ref,
             best_val, best_idx):
    i = pl.program_id(0)
    temp = temp_ref[...]
    logits = logits_ref[...]
    noise = noise_ref[...]
    col = (jax.lax.broadcasted_iota(jnp.int32, (_B, _TC_VC), 1)
           + (i + _TC_OFF) * _TC_VC)
    valid = col < _V
    key = logits - temp * jnp.log(noise + 1e-10)
    key = jnp.where(valid, key, -jnp.inf)
    local_max = jnp.max(key, axis=1, keepdims=True)
    at_max = (key == local_max) & valid
    local_idx = jnp.min(jnp.where(at_max, col, _V), axis=1, keepdims=True)

    @pl.when(i == 0)
    def _():
        best_val[...] = local_max
        best_idx[...] = local_idx

    @pl.when(i > 0)
    def _():
        bv = best_val[...]
        take = local_max > bv
        best_val[...] = jnp.where(take, local_max, bv)
        best_idx[...] = jnp.where(take, local_idx, best_idx[...])

    @pl.when(i == _TC_NCHUNK - 1)
    def _():
        ov_ref[...] = best_val[...]
        oi_ref[...] = best_idx[...]


def _merge_body(v_ref, i_ref, tv_ref, ti_ref, o_ref):
    v = v_ref[...]
    i = i_ref[...]
    tv = tv_ref[...]
    ti = ti_ref[...]
    sv = jnp.max(v, axis=1, keepdims=True)
    si = jnp.min(jnp.where(v == sv, i, _V), axis=1, keepdims=True)
    take = (tv > sv) | ((tv == sv) & (ti < si))
    o_ref[...] = jnp.where(take, ti, si)


@jax.jit
def kernel(logits, temperatures, exp_noise):
    l = logits.astype(jnp.float32)
    t = temperatures.astype(jnp.float32)
    t16 = jnp.broadcast_to(t[:, None], (_B, 16)).reshape(_B * 16)
    pv, pi = _sc_scan(l, exp_noise, t16)
    tv, ti = pl.pallas_call(
        _tc_body,
        grid=(_TC_NCHUNK,),
        in_specs=[
            pl.BlockSpec((_B, 1), lambda i: (0, 0)),
            pl.BlockSpec((_B, _TC_VC), lambda i: (0, i + _TC_OFF)),
            pl.BlockSpec((_B, _TC_VC), lambda i: (0, i + _TC_OFF)),
        ],
        out_specs=[
            pl.BlockSpec((_B, 1), lambda i: (0, 0)),
            pl.BlockSpec((_B, 1), lambda i: (0, 0)),
        ],
        out_shape=[
            jax.ShapeDtypeStruct((_B, 1), jnp.float32),
            jax.ShapeDtypeStruct((_B, 1), jnp.int32),
        ],
        scratch_shapes=[
            pltpu.VMEM((_B, 1), jnp.float32),
            pltpu.VMEM((_B, 1), jnp.int32),
        ],
    )(t.reshape(_B, 1), l, exp_noise)
    out = pl.pallas_call(
        _merge_body,
        out_shape=jax.ShapeDtypeStruct((_B, 1), jnp.int32),
    )(pv.reshape(32, _B).T, pi.reshape(32, _B).T, tv, ti)
    return out.reshape(_B)
